# baseline (speedup 1.0000x reference)
import jax
import jax.numpy as jnp
from jax.experimental import pallas as pl


B, N, M, C = 2, 4096, 4096, 3
BN = 512
K = 8


def _chamfer_body(a_ref, bt_ref, d1_ref, d2_ref):
    nb = pl.program_id(1)
    t = jnp.dot(a_ref[0], bt_ref[0],
                preferred_element_type=jnp.float32,
                precision=jax.lax.Precision.HIGHEST)
    d1_ref[0, 0, :] = jnp.min(t, axis=1)
    part = jnp.min(t, axis=0)

    @pl.when(nb == 0)
    def _init():
        d2_ref[0, 0, :] = part

    @pl.when(nb != 0)
    def _accum():
        d2_ref[0, 0, :] = jnp.minimum(d2_ref[0, 0, :], part)


@jax.jit
def kernel(xyz1, xyz2):
    n1 = jnp.sum(xyz1 * xyz1, axis=-1, keepdims=True)
    n2 = jnp.sum(xyz2 * xyz2, axis=-1, keepdims=True)
    ones1 = jnp.ones_like(n1)
    zeros1 = jnp.zeros((B, N, 3), jnp.float32)
    a_aug = jnp.concatenate([-2.0 * xyz1, ones1, n1, zeros1], axis=-1)
    b_aug = jnp.concatenate([xyz2, n2, jnp.ones_like(n2),
                             jnp.zeros((B, M, 3), jnp.float32)], axis=-1)
    bt = jnp.transpose(b_aug, (0, 2, 1))

    grid = (B, N // BN)
    d1, d2 = pl.pallas_call(
        _chamfer_body,
        grid=grid,
        in_specs=[
            pl.BlockSpec((1, BN, K), lambda b, nb: (b, nb, 0)),
            pl.BlockSpec((1, K, M), lambda b, nb: (b, 0, 0)),
        ],
        out_specs=[
            pl.BlockSpec((1, 1, BN), lambda b, nb: (b, 0, nb)),
            pl.BlockSpec((1, 1, M), lambda b, nb: (b, 0, 0)),
        ],
        out_shape=[
            jax.ShapeDtypeStruct((B, 1, N), jnp.float32),
            jax.ShapeDtypeStruct((B, 1, M), jnp.float32),
        ],
    )(a_aug, bt)
    return d1.reshape(B, N), d2.reshape(B, M)


# speedup vs baseline: 2.8625x; 2.8625x over previous
import jax
import jax.numpy as jnp
from jax.experimental import pallas as pl


B, N, M, C = 2, 4096, 4096, 3
BN = 512
K = 16


def _chamfer_body(a_ref, bt_ref, d1_ref, d2_ref):
    nb = pl.program_id(1)
    t = jnp.dot(a_ref[0], bt_ref[0],
                preferred_element_type=jnp.float32)
    d1_ref[0, 0, :] = jnp.min(t, axis=1)
    part = jnp.min(t, axis=0)

    @pl.when(nb == 0)
    def _init():
        d2_ref[0, 0, :] = part

    @pl.when(nb != 0)
    def _accum():
        d2_ref[0, 0, :] = jnp.minimum(d2_ref[0, 0, :], part)


def _split2(x):
    hi = x.astype(jnp.bfloat16)
    lo = (x - hi.astype(jnp.float32)).astype(jnp.bfloat16)
    return hi, lo


def _split3(x):
    hi = x.astype(jnp.bfloat16)
    r = x - hi.astype(jnp.float32)
    mid = r.astype(jnp.bfloat16)
    lo = (r - mid.astype(jnp.float32)).astype(jnp.bfloat16)
    return hi, mid, lo


@jax.jit
def kernel(xyz1, xyz2):
    f32 = jnp.float32
    n1 = jnp.sum(xyz1 * xyz1, axis=-1, keepdims=True)
    n2 = jnp.sum(xyz2 * xyz2, axis=-1, keepdims=True)
    ahi, alo = _split2(xyz1)
    bhi, blo = _split2(xyz2)
    n1hi, n1mid, n1lo = _split3(n1)
    n2hi, n2mid, n2lo = _split3(n2)
    one1 = jnp.ones_like(n1, jnp.bfloat16)
    one2 = jnp.ones_like(n2, jnp.bfloat16)
    zero1 = jnp.zeros_like(n1, jnp.bfloat16)
    zero2 = jnp.zeros_like(n2, jnp.bfloat16)

    a_aug = jnp.concatenate(
        [-2.0 * ahi, -2.0 * ahi, -2.0 * alo,
         n1hi, n1mid, n1lo, one1, one1, one1, zero1], axis=-1)
    b_aug = jnp.concatenate(
        [bhi, blo, bhi, one2, one2, one2,
         n2hi, n2mid, n2lo, zero2], axis=-1)
    bt = jnp.transpose(b_aug, (0, 2, 1))

    grid = (B, N // BN)
    d1, d2 = pl.pallas_call(
        _chamfer_body,
        grid=grid,
        in_specs=[
            pl.BlockSpec((1, BN, K), lambda b, nb: (b, nb, 0)),
            pl.BlockSpec((1, K, M), lambda b, nb: (b, 0, 0)),
        ],
        out_specs=[
            pl.BlockSpec((1, 1, BN), lambda b, nb: (b, 0, nb)),
            pl.BlockSpec((1, 1, M), lambda b, nb: (b, 0, 0)),
        ],
        out_shape=[
            jax.ShapeDtypeStruct((B, 1, N), f32),
            jax.ShapeDtypeStruct((B, 1, M), f32),
        ],
    )(a_aug, bt)
    return d1.reshape(B, N), d2.reshape(B, M)
